# baseline (speedup 1.0000x reference)
import jax
import jax.numpy as jnp
from jax import lax
from jax.experimental import pallas as pl
from jax.experimental.pallas import tpu as pltpu

_GBLK = 5120
_PBLK = 5000


def _stage_a(cls_ref, tgt_ref, sc_ref):
    x = cls_ref[...]
    xT = jnp.swapaxes(x, 0, 1)
    tgt = tgt_ref[0]
    m = jnp.max(xT, axis=0, keepdims=True)
    e = jnp.exp(xT - m)
    se = jnp.sum(e, axis=0, keepdims=True)
    lse = m + jnp.log(se)
    ids = lax.broadcasted_iota(jnp.int32, xT.shape, 0)
    picked = jnp.sum(jnp.where(ids == tgt, xT, 0.0), axis=0, keepdims=True)
    con = jnp.maximum(lse - picked, 0.0)
    bits = lax.bitcast_convert_type(con, jnp.int32)
    sign = jnp.where(tgt > 0, jnp.int32(-2147483648), jnp.int32(0))
    sc_ref[0] = bits | sign


def _loc_kernel(tgt_ref, op_ref, ot_ref, loc_ref):
    tgt = tgt_ref[0, 0]
    d = op_ref[0] - ot_ref[0]
    ad = jnp.abs(d)
    sl1 = jnp.where(ad < 1.0, 0.5 * d * d, ad - 0.5)
    loc_e = jnp.sum(sl1, axis=0, keepdims=True)
    maskf = (tgt > 0).astype(jnp.float32)
    loc_ref[0] = jnp.sum(maskf * loc_e, axis=(0, 1), keepdims=True)


def _stage_b(sc_ref, loc_ref, out_ref, neg_ref):
    B, P = sc_ref.shape
    bits = sc_ref[...]
    is_pos = bits < 0
    con = lax.bitcast_convert_type(bits & jnp.int32(0x7FFFFFFF), jnp.float32)
    con_neg = jnp.where(is_pos, 0.0, con)
    posf = jnp.sum(is_pos.astype(jnp.float32), axis=1, keepdims=True)
    cm = jnp.sum(jnp.where(is_pos, con, 0.0), axis=1, keepdims=True)
    cs = jnp.sum(con, axis=1, keepdims=True)
    kf = jnp.minimum(3.0 * posf, float(P))
    kcf = jnp.maximum(kf, 1.0)

    neg_ref[...] = cs

    @pl.when(jnp.any(kf < float(P)))
    def _():
        def bbody(_, lohi):
            lo, hi = lohi
            mid = lo + lax.shift_right_logical(hi - lo, 1)
            midf = lax.bitcast_convert_type(mid, jnp.float32)
            c = jnp.sum((con_neg > midf).astype(jnp.float32),
                        axis=1, keepdims=True)
            pred = c < kcf
            return (jnp.where(pred, lo, mid + 1), jnp.where(pred, mid, hi))

        z = jnp.zeros((B, 1), jnp.int32)
        top = jnp.full((B, 1), 0x7F800000, jnp.int32)
        _, hi = lax.fori_loop(0, 31, bbody, (z, top))
        t = lax.bitcast_convert_type(hi, jnp.float32)

        gt = con_neg > t
        cnt_gt = jnp.sum(gt.astype(jnp.float32), axis=1, keepdims=True)
        sum_gt = jnp.sum(jnp.where(gt, con_neg, 0.0), axis=1, keepdims=True)
        mneed = kcf - cnt_gt
        eq = con_neg == t
        iidx = lax.broadcasted_iota(jnp.int32, (B, P), 1)

        def jbody(_, lohi):
            lo, hi2 = lohi
            mid = lo + lax.shift_right_logical(hi2 - lo, 1)
            c = jnp.sum(jnp.where(eq & (iidx < mid), 1.0, 0.0),
                        axis=1, keepdims=True)
            pred = c >= mneed
            return (jnp.where(pred, lo, mid + 1), jnp.where(pred, mid, hi2))

        jz = jnp.zeros((B, 1), jnp.int32)
        jtop = jnp.full((B, 1), P, jnp.int32)
        _, jhi = lax.fori_loop(0, 15, jbody, (jz, jtop))
        sel = eq & (iidx < jhi)
        sum_eq = jnp.sum(jnp.where(sel, con, 0.0), axis=1, keepdims=True)
        neg_ref[...] = sum_gt + sum_eq

    conf = cm + neg_ref[...]
    total = loc_ref[...] + conf
    valid = posf > 0
    per = jnp.where(valid, total / jnp.maximum(posf, 1e-6), 0.0)
    out_ref[...] = jnp.sum(per, axis=0, keepdims=True) / float(B)


def kernel(prior_boxes, classes_preds, offset_preds, offset_targets,
           classes_targets, priors_mask):
    B, P, C = classes_preds.shape
    n = B * P
    ng = n // _GBLK
    cls2 = classes_preds.reshape(n, C)
    tgt32 = classes_targets.astype(jnp.int32)
    tgt3 = tgt32.reshape(ng, 1, _GBLK)

    sc = pl.pallas_call(
        _stage_a,
        grid=(ng,),
        in_specs=[
            pl.BlockSpec((_GBLK, C), lambda g: (g, 0)),
            pl.BlockSpec((1, 1, _GBLK), lambda g: (g, 0, 0)),
        ],
        out_specs=pl.BlockSpec((1, 1, _GBLK), lambda g: (g, 0, 0)),
        out_shape=jax.ShapeDtypeStruct((ng, 1, _GBLK), jnp.int32),
        compiler_params=pltpu.CompilerParams(
            dimension_semantics=("parallel",)),
    )(cls2, tgt3)

    opT = jnp.swapaxes(offset_preds, 1, 2)
    otT = jnp.swapaxes(offset_targets, 1, 2)
    tgt4 = tgt32.reshape(B, 1, 1, P)
    loc = pl.pallas_call(
        _loc_kernel,
        grid=(B,),
        in_specs=[
            pl.BlockSpec((1, 1, 1, P), lambda b: (b, 0, 0, 0)),
            pl.BlockSpec((1, 4, P), lambda b: (b, 0, 0)),
            pl.BlockSpec((1, 4, P), lambda b: (b, 0, 0)),
        ],
        out_specs=pl.BlockSpec((1, 1, 1), lambda b: (b, 0, 0)),
        out_shape=jax.ShapeDtypeStruct((B, 1, 1), jnp.float32),
        compiler_params=pltpu.CompilerParams(
            dimension_semantics=("arbitrary",)),
    )(tgt4, opT, otT)

    out = pl.pallas_call(
        _stage_b,
        out_shape=jax.ShapeDtypeStruct((1, 1), jnp.float32),
        scratch_shapes=[pltpu.VMEM((B, 1), jnp.float32)],
    )(sc.reshape(B, P), loc.reshape(B, 1))
    return out[0, 0]


# speedup vs baseline: 1.0246x; 1.0246x over previous
import jax
import jax.numpy as jnp
from jax import lax
from jax.experimental import pallas as pl
from jax.experimental.pallas import tpu as pltpu

_GBLK = 8000
_PBLK = 5000


def _stage_a(cls_ref, tgt_ref, sc_ref):
    x = cls_ref[...]
    xT = jnp.swapaxes(x, 0, 1)
    tgt = tgt_ref[0]
    m = jnp.max(xT, axis=0, keepdims=True)
    e = jnp.exp(xT - m)
    se = jnp.sum(e, axis=0, keepdims=True)
    lse = m + jnp.log(se)
    ids = lax.broadcasted_iota(jnp.int32, xT.shape, 0)
    picked = jnp.sum(jnp.where(ids == tgt, xT, 0.0), axis=0, keepdims=True)
    con = jnp.maximum(lse - picked, 0.0)
    bits = lax.bitcast_convert_type(con, jnp.int32)
    sign = jnp.where(tgt > 0, jnp.int32(-2147483648), jnp.int32(0))
    sc_ref[0] = bits | sign


def _loc_kernel(tgt_ref, op_ref, ot_ref, loc_ref):
    tgt = tgt_ref[0, 0]
    d = op_ref[0] - ot_ref[0]
    ad = jnp.abs(d)
    sl1 = jnp.where(ad < 1.0, 0.5 * d * d, ad - 0.5)
    loc_e = jnp.sum(sl1, axis=0, keepdims=True)
    maskf = (tgt > 0).astype(jnp.float32)
    loc_ref[0] = jnp.sum(maskf * loc_e, axis=(0, 1), keepdims=True)


def _stage_b(sc_ref, loc_ref, out_ref, neg_ref):
    B, P = sc_ref.shape
    bits = sc_ref[...]
    is_pos = bits < 0
    con = lax.bitcast_convert_type(bits & jnp.int32(0x7FFFFFFF), jnp.float32)
    con_neg = jnp.where(is_pos, 0.0, con)
    posf = jnp.sum(is_pos.astype(jnp.float32), axis=1, keepdims=True)
    cm = jnp.sum(jnp.where(is_pos, con, 0.0), axis=1, keepdims=True)
    cs = jnp.sum(con, axis=1, keepdims=True)
    kf = jnp.minimum(3.0 * posf, float(P))
    kcf = jnp.maximum(kf, 1.0)

    neg_ref[...] = cs

    @pl.when(jnp.any(kf < float(P)))
    def _():
        def bbody(_, lohi):
            lo, hi = lohi
            mid = lo + lax.shift_right_logical(hi - lo, 1)
            midf = lax.bitcast_convert_type(mid, jnp.float32)
            c = jnp.sum((con_neg > midf).astype(jnp.float32),
                        axis=1, keepdims=True)
            pred = c < kcf
            return (jnp.where(pred, lo, mid + 1), jnp.where(pred, mid, hi))

        z = jnp.zeros((B, 1), jnp.int32)
        top = jnp.full((B, 1), 0x7F800000, jnp.int32)
        _, hi = lax.fori_loop(0, 31, bbody, (z, top))
        t = lax.bitcast_convert_type(hi, jnp.float32)

        gt = con_neg > t
        cnt_gt = jnp.sum(gt.astype(jnp.float32), axis=1, keepdims=True)
        sum_gt = jnp.sum(jnp.where(gt, con_neg, 0.0), axis=1, keepdims=True)
        mneed = kcf - cnt_gt
        eq = con_neg == t
        iidx = lax.broadcasted_iota(jnp.int32, (B, P), 1)

        def jbody(_, lohi):
            lo, hi2 = lohi
            mid = lo + lax.shift_right_logical(hi2 - lo, 1)
            c = jnp.sum(jnp.where(eq & (iidx < mid), 1.0, 0.0),
                        axis=1, keepdims=True)
            pred = c >= mneed
            return (jnp.where(pred, lo, mid + 1), jnp.where(pred, mid, hi2))

        jz = jnp.zeros((B, 1), jnp.int32)
        jtop = jnp.full((B, 1), P, jnp.int32)
        _, jhi = lax.fori_loop(0, 15, jbody, (jz, jtop))
        sel = eq & (iidx < jhi)
        sum_eq = jnp.sum(jnp.where(sel, con, 0.0), axis=1, keepdims=True)
        neg_ref[...] = sum_gt + sum_eq

    conf = cm + neg_ref[...]
    total = loc_ref[...] + conf
    valid = posf > 0
    per = jnp.where(valid, total / jnp.maximum(posf, 1e-6), 0.0)
    out_ref[...] = jnp.sum(per, axis=0, keepdims=True) / float(B)


def kernel(prior_boxes, classes_preds, offset_preds, offset_targets,
           classes_targets, priors_mask):
    B, P, C = classes_preds.shape
    n = B * P
    ng = n // _GBLK
    cls2 = classes_preds.reshape(n, C)
    tgt32 = classes_targets.astype(jnp.int32)
    tgt3 = tgt32.reshape(ng, 1, _GBLK)

    sc = pl.pallas_call(
        _stage_a,
        grid=(ng,),
        in_specs=[
            pl.BlockSpec((_GBLK, C), lambda g: (g, 0)),
            pl.BlockSpec((1, 1, _GBLK), lambda g: (g, 0, 0)),
        ],
        out_specs=pl.BlockSpec((1, 1, _GBLK), lambda g: (g, 0, 0)),
        out_shape=jax.ShapeDtypeStruct((ng, 1, _GBLK), jnp.int32),
        compiler_params=pltpu.CompilerParams(
            dimension_semantics=("parallel",)),
    )(cls2, tgt3)

    opT = jnp.swapaxes(offset_preds, 1, 2)
    otT = jnp.swapaxes(offset_targets, 1, 2)
    tgt4 = tgt32.reshape(B, 1, 1, P)
    loc = pl.pallas_call(
        _loc_kernel,
        grid=(B,),
        in_specs=[
            pl.BlockSpec((1, 1, 1, P), lambda b: (b, 0, 0, 0)),
            pl.BlockSpec((1, 4, P), lambda b: (b, 0, 0)),
            pl.BlockSpec((1, 4, P), lambda b: (b, 0, 0)),
        ],
        out_specs=pl.BlockSpec((1, 1, 1), lambda b: (b, 0, 0)),
        out_shape=jax.ShapeDtypeStruct((B, 1, 1), jnp.float32),
        compiler_params=pltpu.CompilerParams(
            dimension_semantics=("arbitrary",)),
    )(tgt4, opT, otT)

    out = pl.pallas_call(
        _stage_b,
        out_shape=jax.ShapeDtypeStruct((1, 1), jnp.float32),
        scratch_shapes=[pltpu.VMEM((B, 1), jnp.float32)],
    )(sc.reshape(B, P), loc.reshape(B, 1))
    return out[0, 0]


# speedup vs baseline: 1.0343x; 1.0094x over previous
import jax
import jax.numpy as jnp
from jax import lax
from jax.experimental import pallas as pl
from jax.experimental.pallas import tpu as pltpu

_GBLK = 10000
_PBLK = 5000


def _stage_a(cls_ref, tgt_ref, sc_ref):
    x = cls_ref[...]
    xT = jnp.swapaxes(x, 0, 1)
    tgt = tgt_ref[0]
    m = jnp.max(xT, axis=0, keepdims=True)
    e = jnp.exp(xT - m)
    se = jnp.sum(e, axis=0, keepdims=True)
    lse = m + jnp.log(se)
    ids = lax.broadcasted_iota(jnp.int32, xT.shape, 0)
    picked = jnp.sum(jnp.where(ids == tgt, xT, 0.0), axis=0, keepdims=True)
    con = jnp.maximum(lse - picked, 0.0)
    bits = lax.bitcast_convert_type(con, jnp.int32)
    sign = jnp.where(tgt > 0, jnp.int32(-2147483648), jnp.int32(0))
    sc_ref[0] = bits | sign


def _loc_kernel(tgt_ref, op_ref, ot_ref, loc_ref):
    tgt = tgt_ref[0, 0]
    d = op_ref[0] - ot_ref[0]
    ad = jnp.abs(d)
    sl1 = jnp.where(ad < 1.0, 0.5 * d * d, ad - 0.5)
    loc_e = jnp.sum(sl1, axis=0, keepdims=True)
    maskf = (tgt > 0).astype(jnp.float32)
    loc_ref[0] = jnp.sum(maskf * loc_e, axis=(0, 1), keepdims=True)


def _stage_b(sc_ref, loc_ref, out_ref, neg_ref):
    B, P = sc_ref.shape
    bits = sc_ref[...]
    is_pos = bits < 0
    con = lax.bitcast_convert_type(bits & jnp.int32(0x7FFFFFFF), jnp.float32)
    con_neg = jnp.where(is_pos, 0.0, con)
    posf = jnp.sum(is_pos.astype(jnp.float32), axis=1, keepdims=True)
    cm = jnp.sum(jnp.where(is_pos, con, 0.0), axis=1, keepdims=True)
    cs = jnp.sum(con, axis=1, keepdims=True)
    kf = jnp.minimum(3.0 * posf, float(P))
    kcf = jnp.maximum(kf, 1.0)

    neg_ref[...] = cs

    @pl.when(jnp.any(kf < float(P)))
    def _():
        def bbody(_, lohi):
            lo, hi = lohi
            mid = lo + lax.shift_right_logical(hi - lo, 1)
            midf = lax.bitcast_convert_type(mid, jnp.float32)
            c = jnp.sum((con_neg > midf).astype(jnp.float32),
                        axis=1, keepdims=True)
            pred = c < kcf
            return (jnp.where(pred, lo, mid + 1), jnp.where(pred, mid, hi))

        z = jnp.zeros((B, 1), jnp.int32)
        top = jnp.full((B, 1), 0x7F800000, jnp.int32)
        _, hi = lax.fori_loop(0, 31, bbody, (z, top))
        t = lax.bitcast_convert_type(hi, jnp.float32)

        gt = con_neg > t
        cnt_gt = jnp.sum(gt.astype(jnp.float32), axis=1, keepdims=True)
        sum_gt = jnp.sum(jnp.where(gt, con_neg, 0.0), axis=1, keepdims=True)
        mneed = kcf - cnt_gt
        eq = con_neg == t
        iidx = lax.broadcasted_iota(jnp.int32, (B, P), 1)

        def jbody(_, lohi):
            lo, hi2 = lohi
            mid = lo + lax.shift_right_logical(hi2 - lo, 1)
            c = jnp.sum(jnp.where(eq & (iidx < mid), 1.0, 0.0),
                        axis=1, keepdims=True)
            pred = c >= mneed
            return (jnp.where(pred, lo, mid + 1), jnp.where(pred, mid, hi2))

        jz = jnp.zeros((B, 1), jnp.int32)
        jtop = jnp.full((B, 1), P, jnp.int32)
        _, jhi = lax.fori_loop(0, 15, jbody, (jz, jtop))
        sel = eq & (iidx < jhi)
        sum_eq = jnp.sum(jnp.where(sel, con, 0.0), axis=1, keepdims=True)
        neg_ref[...] = sum_gt + sum_eq

    conf = cm + neg_ref[...]
    total = loc_ref[...] + conf
    valid = posf > 0
    per = jnp.where(valid, total / jnp.maximum(posf, 1e-6), 0.0)
    out_ref[...] = jnp.sum(per, axis=0, keepdims=True) / float(B)


def kernel(prior_boxes, classes_preds, offset_preds, offset_targets,
           classes_targets, priors_mask):
    B, P, C = classes_preds.shape
    n = B * P
    ng = n // _GBLK
    cls2 = classes_preds.reshape(n, C)
    tgt32 = classes_targets.astype(jnp.int32)
    tgt3 = tgt32.reshape(ng, 1, _GBLK)

    sc = pl.pallas_call(
        _stage_a,
        grid=(ng,),
        in_specs=[
            pl.BlockSpec((_GBLK, C), lambda g: (g, 0)),
            pl.BlockSpec((1, 1, _GBLK), lambda g: (g, 0, 0)),
        ],
        out_specs=pl.BlockSpec((1, 1, _GBLK), lambda g: (g, 0, 0)),
        out_shape=jax.ShapeDtypeStruct((ng, 1, _GBLK), jnp.int32),
        compiler_params=pltpu.CompilerParams(
            dimension_semantics=("parallel",)),
    )(cls2, tgt3)

    opT = jnp.swapaxes(offset_preds, 1, 2)
    otT = jnp.swapaxes(offset_targets, 1, 2)
    tgt4 = tgt32.reshape(B, 1, 1, P)
    loc = pl.pallas_call(
        _loc_kernel,
        grid=(B,),
        in_specs=[
            pl.BlockSpec((1, 1, 1, P), lambda b: (b, 0, 0, 0)),
            pl.BlockSpec((1, 4, P), lambda b: (b, 0, 0)),
            pl.BlockSpec((1, 4, P), lambda b: (b, 0, 0)),
        ],
        out_specs=pl.BlockSpec((1, 1, 1), lambda b: (b, 0, 0)),
        out_shape=jax.ShapeDtypeStruct((B, 1, 1), jnp.float32),
        compiler_params=pltpu.CompilerParams(
            dimension_semantics=("arbitrary",)),
    )(tgt4, opT, otT)

    out = pl.pallas_call(
        _stage_b,
        out_shape=jax.ShapeDtypeStruct((1, 1), jnp.float32),
        scratch_shapes=[pltpu.VMEM((B, 1), jnp.float32)],
    )(sc.reshape(B, P), loc.reshape(B, 1))
    return out[0, 0]
